# baseline (speedup 1.0000x reference)
import functools

import jax
import jax.numpy as jnp
from jax import lax
from jax.experimental import pallas as pl
from jax.experimental.pallas import tpu as pltpu
from jax.experimental.pallas import tpu_sc as plsc

D_MODEL = 128
SCALE = float(D_MODEL) ** 0.5

_NC = 2
_NS = 16
_NW = _NC * _NS

_LANES = 16
_NBUF = 8
_K = 5
_NCHUNK = 4


def _make_kernel(batch: int, seq: int):
    assert batch % _NW == 0
    nch = batch // _NW
    assert nch >= _NBUF

    mesh = plsc.VectorSubcoreMesh(core_axis_name="c", subcore_axis_name="s")

    @functools.partial(
        pl.kernel,
        out_type=jax.ShapeDtypeStruct((batch, seq, D_MODEL), jnp.float32),
        mesh=mesh,
        scratch_types=(
            [pltpu.VMEM((nch, seq), jnp.int32)]
            + [pltpu.VMEM((seq, D_MODEL), jnp.float32)] * _NBUF
            + [pltpu.SemaphoreType.DMA] * (2 * _NBUF)
        ),
    )
    def emb_kernel(x_hbm, table_hbm, out_hbm, idx_v, *bufs_and_sems):
        rows = bufs_and_sems[:_NBUF]
        gsem = bufs_and_sems[_NBUF:2 * _NBUF]
        ssem = bufs_and_sems[2 * _NBUF:]

        wid = lax.axis_index("s") * _NC + lax.axis_index("c")
        row0 = wid * nch
        pltpu.sync_copy(x_hbm.at[pl.ds(row0, nch)], idx_v)

        def start_gather(j, b):
            pltpu.async_copy(table_hbm.at[idx_v.at[j]], rows[b], gsem[b])

        def wait_gather(b):
            pltpu.make_async_copy(
                table_hbm.at[idx_v.at[0]], rows[b], gsem[b]).wait()

        def start_store(j, b):
            pltpu.async_copy(rows[b], out_hbm.at[row0 + j], ssem[b])

        def wait_store(b):
            pltpu.make_async_copy(
                rows[b], out_hbm.at[row0], ssem[b]).wait()

        for b in range(_K):
            start_gather(b, b)

        def _scale(b):
            def scale_row(r, c2):
                for c in range(D_MODEL // _LANES):
                    sl = pl.ds(c * _LANES, _LANES)
                    rows[b][r, sl] = rows[b][r, sl] * SCALE
                return c2
            lax.fori_loop(0, seq, scale_row, 0)

        def outer(o, carry):
            for b in range(_NBUF):
                j = o * _NBUF + b
                jn = j + _K
                bn = (b + _K) % _NBUF

                @pl.when(jn < nch)
                def _():
                    @pl.when(j >= _NBUF - _K)
                    def _():
                        wait_store(bn)
                    start_gather(jn, bn)

                wait_gather(b)
                _scale(b)
                start_store(j, b)
            return carry
        assert nch % _NBUF == 0
        lax.fori_loop(0, nch // _NBUF, outer, 0)

        for b in range(_NBUF):
            wait_store(b)

    return emb_kernel


def kernel(x, table):
    b, s = x.shape
    x = x.astype(jnp.int32)
    bc = b // _NCHUNK
    k = _make_kernel(bc, s)
    outs = [
        k(lax.slice(x, (c * bc, 0), ((c + 1) * bc, s)), table)
        for c in range(_NCHUNK)
    ]
    return jnp.concatenate(outs, axis=0)


# speedup vs baseline: 2.6179x; 2.6179x over previous
import functools

import jax
import jax.numpy as jnp
from jax import lax
from jax.experimental import pallas as pl
from jax.experimental.pallas import tpu as pltpu
from jax.experimental.pallas import tpu_sc as plsc

D_MODEL = 128
SCALE = float(D_MODEL) ** 0.5

_NC = 2
_NS = 16
_NW = _NC * _NS

_LANES = 16
_TB = 8
_NBUF = 2


def _make_kernel(batch: int, seq: int):
    assert batch % (_NW * _TB) == 0
    nch = batch // _NW
    nblk = nch // _TB

    mesh = plsc.VectorSubcoreMesh(core_axis_name="c", subcore_axis_name="s")

    @functools.partial(
        pl.kernel,
        out_type=jax.ShapeDtypeStruct((seq, batch, D_MODEL), jnp.float32),
        mesh=mesh,
        scratch_types=(
            [pltpu.VMEM((nch, seq), jnp.int32)]
            + [pltpu.VMEM((seq, _TB, D_MODEL), jnp.float32)] * _NBUF
            + [pltpu.SemaphoreType.DMA] * (2 * _NBUF)
        ),
    )
    def emb_kernel(x_hbm, table_hbm, out_hbm, idx_v, *bufs_and_sems):
        bufs = bufs_and_sems[:_NBUF]
        gsem = bufs_and_sems[_NBUF:2 * _NBUF]
        ssem = bufs_and_sems[2 * _NBUF:]

        wid = lax.axis_index("s") * _NC + lax.axis_index("c")
        row0 = wid * nch
        pltpu.sync_copy(x_hbm.at[pl.ds(row0, nch)], idx_v)

        def start_gathers(j, b):
            for t in range(_TB):
                pltpu.async_copy(
                    table_hbm.at[idx_v.at[j * _TB + t]],
                    bufs[b].at[:, t],
                    gsem[b],
                )

        def wait_gathers(b):
            for t in range(_TB):
                pltpu.make_async_copy(
                    table_hbm.at[idx_v.at[0]], bufs[b].at[:, t],
                    gsem[b]).wait()

        def start_store(j, b):
            pltpu.async_copy(
                bufs[b],
                out_hbm.at[:, pl.ds(row0 + j * _TB, _TB)],
                ssem[b],
            )

        def wait_store(b):
            pltpu.make_async_copy(
                bufs[b], out_hbm.at[:, pl.ds(row0, _TB)], ssem[b]).wait()

        def _scale(b):
            def scale_row(s, c2):
                for t in range(_TB):
                    for c in range(D_MODEL // _LANES):
                        sl = pl.ds(c * _LANES, _LANES)
                        bufs[b][s, t, sl] = bufs[b][s, t, sl] * SCALE
                return c2
            lax.fori_loop(0, seq, scale_row, 0)

        start_gathers(0, 0)

        def outer(o, carry):
            for b in range(_NBUF):
                j = o * _NBUF + b
                bn = (b + 1) % _NBUF
                @pl.when(j + 1 < nblk)
                def _():
                    @pl.when(j >= _NBUF - 1)
                    def _():
                        wait_store(bn)
                    start_gathers(j + 1, bn)
                wait_gathers(b)
                _scale(b)
                start_store(j, b)
            return carry

        assert nblk % _NBUF == 0
        lax.fori_loop(0, nblk // _NBUF, outer, 0)
        for b in range(min(_NBUF, nblk)):
            wait_store(b)

    return emb_kernel


def kernel(x, table):
    b, s = x.shape
    out = _make_kernel(b, s)(x.astype(jnp.int32), table)
    return jnp.transpose(out, (1, 0, 2))


# speedup vs baseline: 2.9708x; 1.1348x over previous
import functools

import jax
import jax.numpy as jnp
from jax import lax
from jax.experimental import pallas as pl
from jax.experimental.pallas import tpu as pltpu
from jax.experimental.pallas import tpu_sc as plsc

D_MODEL = 128
SCALE = float(D_MODEL) ** 0.5

_NC = 2
_NS = 16
_NW = _NC * _NS

_LANES = 16
_TB = 8
_SS = 2
_NBUF = 4
_K = 2


def _make_kernel(batch: int, seq: int):
    assert batch % (_NW * _TB) == 0
    assert seq % _SS == 0
    nch = batch // _NW
    nblk = nch // _TB
    sb = seq // _SS
    nsub = nblk * _SS

    mesh = plsc.VectorSubcoreMesh(core_axis_name="c", subcore_axis_name="s")

    @functools.partial(
        pl.kernel,
        out_type=jax.ShapeDtypeStruct((seq, batch, D_MODEL), jnp.float32),
        mesh=mesh,
        scratch_types=(
            [pltpu.VMEM((nch, seq), jnp.int32)]
            + [pltpu.VMEM((sb, _TB, D_MODEL), jnp.float32)] * _NBUF
            + [pltpu.SemaphoreType.DMA] * (2 * _NBUF)
        ),
    )
    def emb_kernel(x_hbm, table_hbm, out_hbm, idx_v, *bufs_and_sems):
        bufs = bufs_and_sems[:_NBUF]
        gsem = bufs_and_sems[_NBUF:2 * _NBUF]
        ssem = bufs_and_sems[2 * _NBUF:]

        wid = lax.axis_index("s") * _NC + lax.axis_index("c")
        row0 = wid * nch
        pltpu.sync_copy(x_hbm.at[pl.ds(row0, nch)], idx_v)

        def start_gathers(j, h, b):
            for t in range(_TB):
                pltpu.async_copy(
                    table_hbm.at[idx_v.at[j * _TB + t, pl.ds(h * sb, sb)]],
                    bufs[b].at[:, t],
                    gsem[b],
                )

        def wait_gathers(b):
            for t in range(_TB):
                pltpu.make_async_copy(
                    table_hbm.at[idx_v.at[0, pl.ds(0, sb)]],
                    bufs[b].at[:, t], gsem[b]).wait()

        def start_store(j, h, b):
            pltpu.async_copy(
                bufs[b],
                out_hbm.at[pl.ds(h * sb, sb), pl.ds(row0 + j * _TB, _TB)],
                ssem[b],
            )

        def wait_store(b):
            pltpu.make_async_copy(
                bufs[b], out_hbm.at[pl.ds(0, sb), pl.ds(row0, _TB)],
                ssem[b]).wait()

        def _scale(b):
            def scale_row(s, c2):
                for t in range(_TB):
                    for c in range(D_MODEL // _LANES):
                        sl = pl.ds(c * _LANES, _LANES)
                        bufs[b][s, t, sl] = bufs[b][s, t, sl] * SCALE
                return c2
            lax.fori_loop(0, sb, scale_row, 0)

        for c in range(_K):
            start_gathers(c // _SS, c % _SS, c % _NBUF)

        def outer(o, carry):
            for u in range(_NBUF):
                c = o * _NBUF + u
                cn = c + _K
                un = (u + _K) % _NBUF
                @pl.when(cn < nsub)
                def _():
                    @pl.when(c >= _NBUF - _K)
                    def _():
                        wait_store(un)
                    start_gathers(o * _SS + (u + _K) // _SS, (u + _K) % _SS,
                                  un)
                wait_gathers(u)
                _scale(u)
                start_store(o * _SS + u // _SS, u % _SS, u)
            return carry

        assert nsub % _NBUF == 0
        assert _NBUF == _SS * 2 and _K == _SS
        lax.fori_loop(0, nsub // _NBUF, outer, 0)
        for b in range(min(_NBUF, nsub)):
            wait_store(b)

    return emb_kernel


def kernel(x, table):
    b, s = x.shape
    out = _make_kernel(b, s)(x.astype(jnp.int32), table)
    return jnp.transpose(out, (1, 0, 2))
